# baseline (speedup 1.0000x reference)
import jax
import jax.numpy as jnp
from jax import lax
from jax.experimental import pallas as pl
from jax.experimental.pallas import tpu as pltpu
from jax.experimental.pallas import tpu_sc as plsc

VOCAB = 1_000_000
D = 64
B = 4096
CTX = 20
NCORES = 2
NSUB = 16
NW = NCORES * NSUB
ROWS_PER_W = B * CTX // NW
CHUNK = 80
NCHUNKS = ROWS_PER_W // CHUNK
VB = 8192


def _sc_gather_sum(idx_hbm, table_hbm, out_hbm, idx_v, rows_v, acc_v, sem):
    c = lax.axis_index("c")
    s = lax.axis_index("s")
    wid = s * NCORES + c
    base = wid * ROWS_PER_W
    zero = jnp.zeros((16,), jnp.float32)
    for l in range(CTX):
        for j in range(D // 16):
            acc_v[l, pl.ds(j * 16, 16)] = zero

    def chunk_body(ci, carry):
        pltpu.sync_copy(idx_hbm.at[pl.ds(base + ci * CHUNK, CHUNK)], idx_v)
        pltpu.async_copy(table_hbm.at[idx_v], rows_v, sem).wait()
        for bb in range(CHUNK // CTX):
            for l in range(CTX):
                for j in range(D // 16):
                    plsc.addupdate(
                        acc_v.at[l, pl.ds(j * 16, 16)],
                        rows_v[bb * CTX + l, pl.ds(j * 16, 16)],
                    )
        return carry

    lax.fori_loop(0, NCHUNKS, chunk_body, 0)
    pltpu.sync_copy(acc_v, out_hbm.at[wid])


def _sc_partial_sums(idx_flat, emb_table):
    mesh = plsc.VectorSubcoreMesh(core_axis_name="c", subcore_axis_name="s")
    return pl.kernel(
        _sc_gather_sum,
        out_type=jax.ShapeDtypeStruct((NW, CTX, D), jnp.float32),
        mesh=mesh,
        scratch_types=[
            pltpu.VMEM((CHUNK,), jnp.int32),
            pltpu.VMEM((CHUNK, D), jnp.float32),
            pltpu.VMEM((CTX, D), jnp.float32),
            pltpu.SemaphoreType.DMA,
        ],
        compiler_params=pltpu.CompilerParams(use_tc_tiling_on_sc=False),
    )(idx_flat, emb_table)


def _mm_body(part_ref, w_ref, b_ref, out_ref):
    combined = jnp.sum(part_ref[...], axis=0) * (1.0 / B)
    out_ref[...] = (
        lax.dot_general(
            combined,
            w_ref[...],
            (((1,), (1,)), ((), ())),
            preferred_element_type=jnp.float32,
        )
        + b_ref[...]
    )


def _tc_matmul(partials, W, b2d):
    return pl.pallas_call(
        _mm_body,
        grid=(pl.cdiv(VOCAB, VB),),
        in_specs=[
            pl.BlockSpec((NW, CTX, D), lambda i: (0, 0, 0)),
            pl.BlockSpec((VB, D), lambda i: (i, 0)),
            pl.BlockSpec((1, VB), lambda i: (0, i)),
        ],
        out_specs=pl.BlockSpec((CTX, VB), lambda i: (0, i)),
        out_shape=jax.ShapeDtypeStruct((CTX, VOCAB), jnp.float32),
    )(partials, W, b2d)


def kernel(context_idxs, emb_table, W, b):
    idx_flat = context_idxs.reshape(-1).astype(jnp.int32)
    partials = _sc_partial_sums(idx_flat, emb_table)
    return _tc_matmul(partials, W, b.reshape(1, VOCAB))


# speedup vs baseline: 1.3965x; 1.3965x over previous
import jax
import jax.numpy as jnp
from jax import lax
from jax.experimental import pallas as pl
from jax.experimental.pallas import tpu as pltpu
from jax.experimental.pallas import tpu_sc as plsc

VOCAB = 1_000_000
D = 64
B = 4096
CTX = 20
NCORES = 2
NSUB = 16
NW = NCORES * NSUB
ROWS_PER_W = B * CTX // NW
CHUNK = 80
NCHUNKS = ROWS_PER_W // CHUNK
VB = 8192


def _sc_gather_sum(idx_hbm, table_hbm, out_hbm, idx_v, rows_v, acc_v, sem):
    c = lax.axis_index("c")
    s = lax.axis_index("s")
    wid = s * NCORES + c
    base = wid * ROWS_PER_W
    zero = jnp.zeros((16,), jnp.float32)
    for l in range(CTX):
        for j in range(D // 16):
            acc_v[l, pl.ds(j * 16, 16)] = zero

    def chunk_body(ci, carry):
        pltpu.sync_copy(idx_hbm.at[pl.ds(base + ci * CHUNK, CHUNK)], idx_v)
        pltpu.async_copy(table_hbm.at[idx_v], rows_v, sem).wait()
        for bb in range(CHUNK // CTX):
            for l in range(CTX):
                for j in range(D // 16):
                    plsc.addupdate(
                        acc_v.at[l, pl.ds(j * 16, 16)],
                        rows_v[bb * CTX + l, pl.ds(j * 16, 16)],
                    )
        return carry

    lax.fori_loop(0, NCHUNKS, chunk_body, 0)
    pltpu.sync_copy(acc_v, out_hbm.at[wid])


def _sc_partial_sums(idx_flat, emb_table):
    mesh = plsc.VectorSubcoreMesh(core_axis_name="c", subcore_axis_name="s")
    return pl.kernel(
        _sc_gather_sum,
        out_type=jax.ShapeDtypeStruct((NW, CTX, D), jnp.float32),
        mesh=mesh,
        scratch_types=[
            pltpu.VMEM((CHUNK,), jnp.int32),
            pltpu.VMEM((CHUNK, D), jnp.float32),
            pltpu.VMEM((CTX, D), jnp.float32),
            pltpu.SemaphoreType.DMA,
        ],
        compiler_params=pltpu.CompilerParams(use_tc_tiling_on_sc=False),
    )(idx_flat, emb_table)


def _mm_body(part_ref, wt_ref, b_ref, out_ref):
    combined = jnp.sum(part_ref[...], axis=0) * (1.0 / B)
    out_ref[...] = (
        lax.dot_general(
            combined,
            wt_ref[...],
            (((1,), (0,)), ((), ())),
            preferred_element_type=jnp.float32,
        )
        + b_ref[...]
    )


def _tc_matmul(partials, Wt, b2d):
    return pl.pallas_call(
        _mm_body,
        grid=(pl.cdiv(VOCAB, VB),),
        in_specs=[
            pl.BlockSpec((NW, CTX, D), lambda i: (0, 0, 0)),
            pl.BlockSpec((D, VB), lambda i: (0, i)),
            pl.BlockSpec((1, VB), lambda i: (0, i)),
        ],
        out_specs=pl.BlockSpec((CTX, VB), lambda i: (0, i)),
        out_shape=jax.ShapeDtypeStruct((CTX, VOCAB), jnp.float32),
    )(partials, Wt, b2d)


def kernel(context_idxs, emb_table, W, b):
    idx_flat = context_idxs.reshape(-1).astype(jnp.int32)
    partials = _sc_partial_sums(idx_flat, emb_table)
    return _tc_matmul(partials, W.T, b.reshape(1, VOCAB))


# speedup vs baseline: 1.4734x; 1.0550x over previous
import jax
import jax.numpy as jnp
from jax import lax
from jax.experimental import pallas as pl
from jax.experimental.pallas import tpu as pltpu
from jax.experimental.pallas import tpu_sc as plsc

VOCAB = 1_000_000
D = 64
B = 4096
CTX = 20
NCORES = 2
NSUB = 16
NW = NCORES * NSUB
ROWS_PER_W = B * CTX // NW
CHUNK = 80
NCHUNKS = ROWS_PER_W // CHUNK
VB = 32768


def _sc_gather_sum(idx_hbm, table_hbm, out_hbm, idx_v, rows_v, acc_v, sem):
    c = lax.axis_index("c")
    s = lax.axis_index("s")
    wid = s * NCORES + c
    base = wid * ROWS_PER_W
    zero = jnp.zeros((16,), jnp.float32)
    for l in range(CTX):
        for j in range(D // 16):
            acc_v[l, pl.ds(j * 16, 16)] = zero

    def chunk_body(ci, carry):
        pltpu.sync_copy(idx_hbm.at[pl.ds(base + ci * CHUNK, CHUNK)], idx_v)
        pltpu.async_copy(table_hbm.at[idx_v], rows_v, sem).wait()
        for bb in range(CHUNK // CTX):
            for l in range(CTX):
                for j in range(D // 16):
                    plsc.addupdate(
                        acc_v.at[l, pl.ds(j * 16, 16)],
                        rows_v[bb * CTX + l, pl.ds(j * 16, 16)],
                    )
        return carry

    lax.fori_loop(0, NCHUNKS, chunk_body, 0)
    pltpu.sync_copy(acc_v, out_hbm.at[wid])


def _sc_partial_sums(idx_flat, emb_table):
    mesh = plsc.VectorSubcoreMesh(core_axis_name="c", subcore_axis_name="s")
    return pl.kernel(
        _sc_gather_sum,
        out_type=jax.ShapeDtypeStruct((NW, CTX, D), jnp.float32),
        mesh=mesh,
        scratch_types=[
            pltpu.VMEM((CHUNK,), jnp.int32),
            pltpu.VMEM((CHUNK, D), jnp.float32),
            pltpu.VMEM((CTX, D), jnp.float32),
            pltpu.SemaphoreType.DMA,
        ],
        compiler_params=pltpu.CompilerParams(use_tc_tiling_on_sc=False),
    )(idx_flat, emb_table)


def _mm_body(part_ref, wt_ref, b_ref, out_ref):
    combined = jnp.sum(part_ref[...], axis=0) * (1.0 / B)
    out_ref[...] = (
        lax.dot_general(
            combined,
            wt_ref[...],
            (((1,), (0,)), ((), ())),
            preferred_element_type=jnp.float32,
        )
        + b_ref[...]
    )


def _tc_matmul(partials, Wt, b2d):
    return pl.pallas_call(
        _mm_body,
        grid=(pl.cdiv(VOCAB, VB),),
        in_specs=[
            pl.BlockSpec((NW, CTX, D), lambda i: (0, 0, 0)),
            pl.BlockSpec((D, VB), lambda i: (0, i)),
            pl.BlockSpec((1, VB), lambda i: (0, i)),
        ],
        out_specs=pl.BlockSpec((CTX, VB), lambda i: (0, i)),
        out_shape=jax.ShapeDtypeStruct((CTX, VOCAB), jnp.float32),
    )(partials, Wt, b2d)


def kernel(context_idxs, emb_table, W, b):
    idx_flat = context_idxs.reshape(-1).astype(jnp.int32)
    partials = _sc_partial_sums(idx_flat, emb_table)
    return _tc_matmul(partials, W.T, b.reshape(1, VOCAB))
